# baseline (speedup 1.0000x reference)
import functools

import jax
import jax.numpy as jnp
from jax import lax
from jax.experimental import pallas as pl
from jax.experimental.pallas import tpu as pltpu
from jax.experimental.pallas import tpu_sc as plsc

N = 10000
E = 320000
D = 128

NC = 2
NS = 16
NW = NC * NS

G = 128
K = 80
CK = 16
E_PAD = NW * K * G
N_PAD = 10240
ROWS_PER_TILE = N_PAD // NS


def _segment_sum_sc(m_pad, src_r, dst_r):
    mesh = plsc.VectorSubcoreMesh(core_axis_name="c", subcore_axis_name="s")

    @functools.partial(
        pl.kernel,
        out_type=jax.ShapeDtypeStruct((NC, N_PAD, D), jnp.float32),
        mesh=mesh,
        scratch_types=[
            pltpu.VMEM((CK, G), jnp.int32),
            pltpu.VMEM((CK, G), jnp.int32),
            pltpu.VMEM((G, D), jnp.float32),
            pltpu.VMEM((G, D), jnp.float32),
            pltpu.VMEM_SHARED((N_PAD, D), jnp.float32),
            pltpu.SemaphoreType.DMA,
            pltpu.SemaphoreType.DMA,
            pltpu.SemaphoreType.DMA,
            pltpu.SemaphoreType.DMA,
        ],
    )
    def k(m_hbm, src_hbm, dst_hbm, out_hbm, src_v, dst_v, bufa, bufb, acc,
          sema, semb, ssema, ssemb):
        cid = lax.axis_index("c")
        sid = lax.axis_index("s")
        wid = cid * NS + sid

        @pl.loop(0, G)
        def _(i):
            @pl.loop(0, D, step=16)
            def _(j):
                bufa.at[pl.ds(i, 1), pl.ds(j, 16)][...] = jnp.zeros(
                    (1, 16), jnp.float32)

        row0 = sid * ROWS_PER_TILE

        @pl.loop(0, ROWS_PER_TILE, step=G)
        def _(r):
            pltpu.sync_copy(bufa, acc.at[pl.ds(row0 + r, G)])

        plsc.subcore_barrier()

        def g_wait(j, buf, sem):
            pltpu.make_async_copy(m_hbm.at[src_v.at[j]], buf, sem).wait()

        def s_start(j, buf, sem):
            pltpu.async_copy(buf, acc.at[dst_v.at[j]], sem, add=True)

        def s_wait(j, buf, sem):
            pltpu.make_async_copy(buf, acc.at[dst_v.at[j]], sem).wait()

        @pl.loop(0, K, step=CK)
        def _(c):
            pltpu.sync_copy(src_hbm.at[pl.ds(wid * K + c, CK)], src_v)
            pltpu.sync_copy(dst_hbm.at[pl.ds(wid * K + c, CK)], dst_v)

            pltpu.async_copy(m_hbm.at[src_v.at[0]], bufa, sema)
            pltpu.async_copy(m_hbm.at[src_v.at[1]], bufb, semb)

            @pl.loop(0, CK - 2, step=2)
            def _(j):
                g_wait(j, bufa, sema)
                s_start(j, bufa, ssema)
                g_wait(j + 1, bufb, semb)
                s_start(j + 1, bufb, ssemb)
                s_wait(j, bufa, ssema)
                pltpu.async_copy(m_hbm.at[src_v.at[j + 2]], bufa, sema)
                s_wait(j + 1, bufb, ssemb)
                pltpu.async_copy(m_hbm.at[src_v.at[j + 3]], bufb, semb)

            g_wait(CK - 2, bufa, sema)
            s_start(CK - 2, bufa, ssema)
            g_wait(CK - 1, bufb, semb)
            s_start(CK - 1, bufb, ssemb)
            s_wait(CK - 2, bufa, ssema)
            s_wait(CK - 1, bufb, ssemb)

        plsc.subcore_barrier()

        @pl.loop(0, ROWS_PER_TILE, step=G)
        def _(r):
            pltpu.sync_copy(acc.at[pl.ds(row0 + r, G)], bufb)
            pltpu.sync_copy(bufb, out_hbm.at[cid, pl.ds(row0 + r, G)])

    return k(m_pad, src_r, dst_r)


_BM = 256


def _dot_t(a, b):
    return lax.dot_general(a, b, (((1,), (1,)), ((), ())),
                           preferred_element_type=jnp.float32)


def _matmul_t(x, w):
    def body(x_ref, w_ref, o_ref):
        o_ref[...] = _dot_t(x_ref[...], w_ref[...])

    return pl.pallas_call(
        body,
        grid=(N_PAD // _BM,),
        in_specs=[
            pl.BlockSpec((_BM, D), lambda i: (i, 0)),
            pl.BlockSpec((D, D), lambda i: (0, 0)),
        ],
        out_specs=pl.BlockSpec((_BM, D), lambda i: (i, 0)),
        out_shape=jax.ShapeDtypeStruct((N_PAD, D), jnp.float32),
    )(x, w)


def _combine_mid(p0, p1, x, w_root, b, w2_rel):
    def body(p0_ref, p1_ref, x_ref, wr_ref, b_ref, w2_ref, h_ref, m2_ref):
        h = p0_ref[...] + p1_ref[...] + b_ref[...] + _dot_t(
            x_ref[...], wr_ref[...])
        h = jnp.maximum(h, 0.0)
        h_ref[...] = h
        m2_ref[...] = _dot_t(h, w2_ref[...])

    return pl.pallas_call(
        body,
        grid=(N_PAD // _BM,),
        in_specs=[
            pl.BlockSpec((_BM, D), lambda i: (i, 0)),
            pl.BlockSpec((_BM, D), lambda i: (i, 0)),
            pl.BlockSpec((_BM, D), lambda i: (i, 0)),
            pl.BlockSpec((D, D), lambda i: (0, 0)),
            pl.BlockSpec((1, D), lambda i: (0, 0)),
            pl.BlockSpec((D, D), lambda i: (0, 0)),
        ],
        out_specs=[
            pl.BlockSpec((_BM, D), lambda i: (i, 0)),
            pl.BlockSpec((_BM, D), lambda i: (i, 0)),
        ],
        out_shape=[
            jax.ShapeDtypeStruct((N_PAD, D), jnp.float32),
            jax.ShapeDtypeStruct((N_PAD, D), jnp.float32),
        ],
    )(p0, p1, x, w_root, b, w2_rel)


def _combine_last(p0, p1, h, w_root, b):
    def body(p0_ref, p1_ref, h_ref, wr_ref, b_ref, o_ref):
        o_ref[...] = p0_ref[...] + p1_ref[...] + b_ref[...] + _dot_t(
            h_ref[...], wr_ref[...])

    return pl.pallas_call(
        body,
        grid=(N_PAD // _BM,),
        in_specs=[
            pl.BlockSpec((_BM, D), lambda i: (i, 0)),
            pl.BlockSpec((_BM, D), lambda i: (i, 0)),
            pl.BlockSpec((_BM, D), lambda i: (i, 0)),
            pl.BlockSpec((D, D), lambda i: (0, 0)),
            pl.BlockSpec((1, D), lambda i: (0, 0)),
        ],
        out_specs=pl.BlockSpec((_BM, D), lambda i: (i, 0)),
        out_shape=jax.ShapeDtypeStruct((N_PAD, D), jnp.float32),
    )(p0, p1, h, w_root, b)


def kernel(x, edge_index, W1_rel, b1_rel, W1_root, W2_rel, b2_rel, W2_root):
    x_pad = jnp.pad(x, ((0, N_PAD - N), (0, 0)))
    pad = jnp.arange(E_PAD - E, dtype=jnp.int32)
    src = jnp.concatenate([edge_index[0], pad % N]).reshape(NW * K, G)
    dst = jnp.concatenate([edge_index[1],
                           N + pad % (N_PAD - N)]).reshape(NW * K, G)

    m1 = _matmul_t(x_pad, W1_rel)
    p1 = _segment_sum_sc(m1, src, dst)
    h1, m2 = _combine_mid(p1[0], p1[1], x_pad, W1_root,
                          b1_rel.reshape(1, D), W2_rel)
    p2 = _segment_sum_sc(m2, src, dst)
    out = _combine_last(p2[0], p2[1], h1, W2_root, b2_rel.reshape(1, D))
    return out[:N]


# speedup vs baseline: 1.1774x; 1.1774x over previous
import functools

import jax
import jax.numpy as jnp
from jax import lax
from jax.experimental import pallas as pl
from jax.experimental.pallas import tpu as pltpu
from jax.experimental.pallas import tpu_sc as plsc

N = 10000
E = 320000
D = 128

NC = 2
NS = 16
NW = NC * NS

G = 128
K = 80
CK = 16
E_PAD = NW * K * G
N_PAD = 10240
ROWS_PER_TILE = N_PAD // NS


def _segment_sum_sc(m_pad, src_r, dst_r):
    mesh = plsc.VectorSubcoreMesh(core_axis_name="c", subcore_axis_name="s")

    @functools.partial(
        pl.kernel,
        out_type=jax.ShapeDtypeStruct((NC, N_PAD, D), jnp.float32),
        mesh=mesh,
        scratch_types=[
            pltpu.VMEM((CK, G), jnp.int32),
            pltpu.VMEM((CK, G), jnp.int32),
            pltpu.VMEM((G, D), jnp.float32),
            pltpu.VMEM((G, D), jnp.float32),
            pltpu.VMEM_SHARED((N_PAD, D), jnp.float32),
            pltpu.SemaphoreType.DMA,
            pltpu.SemaphoreType.DMA,
            pltpu.SemaphoreType.DMA,
            pltpu.SemaphoreType.DMA,
        ],
    )
    def k(m_hbm, src_hbm, dst_hbm, out_hbm, src_v, dst_v, bufa, bufb, acc,
          sema, semb, ssema, ssemb):
        cid = lax.axis_index("c")
        sid = lax.axis_index("s")
        wid = cid * NS + sid

        @pl.loop(0, G)
        def _(i):
            @pl.loop(0, D, step=16)
            def _(j):
                bufa.at[pl.ds(i, 1), pl.ds(j, 16)][...] = jnp.zeros(
                    (1, 16), jnp.float32)

        row0 = sid * ROWS_PER_TILE

        @pl.loop(0, ROWS_PER_TILE, step=G)
        def _(r):
            pltpu.sync_copy(bufa, acc.at[pl.ds(row0 + r, G)])

        plsc.subcore_barrier()

        @pl.loop(0, K, step=CK)
        def _(c):
            pltpu.sync_copy(src_hbm.at[pl.ds(wid * K + c, CK)], src_v)
            pltpu.sync_copy(dst_hbm.at[pl.ds(wid * K + c, CK)], dst_v)

            pltpu.async_copy(m_hbm.at[src_v.at[0]], bufa, sema)

            @pl.loop(0, CK - 2, step=2)
            def _(j):
                pltpu.async_copy(m_hbm.at[src_v.at[j + 1]], bufb, semb)
                pltpu.make_async_copy(m_hbm.at[src_v.at[j]], bufa,
                                      sema).wait()
                pltpu.sync_copy(bufa, acc.at[dst_v.at[j]], add=True)
                pltpu.async_copy(m_hbm.at[src_v.at[j + 2]], bufa, sema)
                pltpu.make_async_copy(m_hbm.at[src_v.at[j + 1]], bufb,
                                      semb).wait()
                pltpu.sync_copy(bufb, acc.at[dst_v.at[j + 1]], add=True)

            pltpu.async_copy(m_hbm.at[src_v.at[CK - 1]], bufb, semb)
            pltpu.make_async_copy(m_hbm.at[src_v.at[CK - 2]], bufa,
                                  sema).wait()
            pltpu.sync_copy(bufa, acc.at[dst_v.at[CK - 2]], add=True)
            pltpu.make_async_copy(m_hbm.at[src_v.at[CK - 1]], bufb,
                                  semb).wait()
            pltpu.sync_copy(bufb, acc.at[dst_v.at[CK - 1]], add=True)

        plsc.subcore_barrier()

        @pl.loop(0, ROWS_PER_TILE, step=G)
        def _(r):
            pltpu.sync_copy(acc.at[pl.ds(row0 + r, G)], bufb)
            pltpu.sync_copy(bufb, out_hbm.at[cid, pl.ds(row0 + r, G)])

    return k(m_pad, src_r, dst_r)


_BM = 256


def _dot_t(a, b):
    return lax.dot_general(a, b, (((1,), (1,)), ((), ())),
                           preferred_element_type=jnp.float32)


def _matmul_t(x, w):
    def body(x_ref, w_ref, o_ref):
        o_ref[...] = _dot_t(x_ref[...], w_ref[...])

    return pl.pallas_call(
        body,
        grid=(N_PAD // _BM,),
        in_specs=[
            pl.BlockSpec((_BM, D), lambda i: (i, 0)),
            pl.BlockSpec((D, D), lambda i: (0, 0)),
        ],
        out_specs=pl.BlockSpec((_BM, D), lambda i: (i, 0)),
        out_shape=jax.ShapeDtypeStruct((N_PAD, D), jnp.float32),
    )(x, w)


def _combine_mid(p0, p1, x, w_root, b, w2_rel):
    def body(p0_ref, p1_ref, x_ref, wr_ref, b_ref, w2_ref, h_ref, m2_ref):
        h = p0_ref[...] + p1_ref[...] + b_ref[...] + _dot_t(
            x_ref[...], wr_ref[...])
        h = jnp.maximum(h, 0.0)
        h_ref[...] = h
        m2_ref[...] = _dot_t(h, w2_ref[...])

    return pl.pallas_call(
        body,
        grid=(N_PAD // _BM,),
        in_specs=[
            pl.BlockSpec((_BM, D), lambda i: (i, 0)),
            pl.BlockSpec((_BM, D), lambda i: (i, 0)),
            pl.BlockSpec((_BM, D), lambda i: (i, 0)),
            pl.BlockSpec((D, D), lambda i: (0, 0)),
            pl.BlockSpec((1, D), lambda i: (0, 0)),
            pl.BlockSpec((D, D), lambda i: (0, 0)),
        ],
        out_specs=[
            pl.BlockSpec((_BM, D), lambda i: (i, 0)),
            pl.BlockSpec((_BM, D), lambda i: (i, 0)),
        ],
        out_shape=[
            jax.ShapeDtypeStruct((N_PAD, D), jnp.float32),
            jax.ShapeDtypeStruct((N_PAD, D), jnp.float32),
        ],
    )(p0, p1, x, w_root, b, w2_rel)


def _combine_last(p0, p1, h, w_root, b):
    def body(p0_ref, p1_ref, h_ref, wr_ref, b_ref, o_ref):
        o_ref[...] = p0_ref[...] + p1_ref[...] + b_ref[...] + _dot_t(
            h_ref[...], wr_ref[...])

    return pl.pallas_call(
        body,
        grid=(N_PAD // _BM,),
        in_specs=[
            pl.BlockSpec((_BM, D), lambda i: (i, 0)),
            pl.BlockSpec((_BM, D), lambda i: (i, 0)),
            pl.BlockSpec((_BM, D), lambda i: (i, 0)),
            pl.BlockSpec((D, D), lambda i: (0, 0)),
            pl.BlockSpec((1, D), lambda i: (0, 0)),
        ],
        out_specs=pl.BlockSpec((_BM, D), lambda i: (i, 0)),
        out_shape=jax.ShapeDtypeStruct((N_PAD, D), jnp.float32),
    )(p0, p1, h, w_root, b)


def kernel(x, edge_index, W1_rel, b1_rel, W1_root, W2_rel, b2_rel, W2_root):
    x_pad = jnp.pad(x, ((0, N_PAD - N), (0, 0)))
    pad = jnp.arange(E_PAD - E, dtype=jnp.int32)
    src = jnp.concatenate([edge_index[0], pad % N]).reshape(NW * K, G)
    dst = jnp.concatenate([edge_index[1],
                           N + pad % (N_PAD - N)]).reshape(NW * K, G)

    m1 = _matmul_t(x_pad, W1_rel)
    p1 = _segment_sum_sc(m1, src, dst)
    h1, m2 = _combine_mid(p1[0], p1[1], x_pad, W1_root,
                          b1_rel.reshape(1, D), W2_rel)
    p2 = _segment_sum_sc(m2, src, dst)
    out = _combine_last(p2[0], p2[1], h1, W2_root, b2_rel.reshape(1, D))
    return out[:N]
